# baseline (speedup 1.0000x reference)
import jax
import jax.numpy as jnp
from jax.experimental import pallas as pl


def _add_body(x_ref, pos_ref, o_ref):
    o_ref[...] = x_ref[...] + pos_ref[...]


def kernel(patch, pos_table):
    B, P, D = patch.shape
    PD = P * D
    x = patch.reshape(B, PD)
    pos = pos_table.reshape(1, PD)
    BB = 8
    out = pl.pallas_call(
        _add_body,
        grid=(B // BB,),
        in_specs=[
            pl.BlockSpec((BB, PD), lambda i: (i, 0)),
            pl.BlockSpec((1, PD), lambda i: (0, 0)),
        ],
        out_specs=pl.BlockSpec((BB, PD), lambda i: (i, 0)),
        out_shape=jax.ShapeDtypeStruct((B, PD), jnp.float32),
    )(x, pos)
    return out.reshape(B, P, D)
